# Initial kernel scaffold; baseline (speedup 1.0000x reference)
import jax
import jax.numpy as jnp
from jax.experimental import pallas as pl


def kernel(input):
    raise NotImplementedError("write your pallas kernel here")


# speedup vs baseline: 20.3657x; 20.3657x over previous
import jax
import jax.numpy as jnp
from jax.experimental import pallas as pl

_BUDGET = 1.0 - 0.05

_N_BISECT = 14
_N_CORRECT = 2


def _sparsegen_block(x_ref, o_ref):
    z = x_ref[...]
    rowmax = jnp.max(z, axis=1, keepdims=True)
    hi = rowmax
    lo = rowmax - _BUDGET
    for _ in range(_N_BISECT):
        mid = 0.5 * (lo + hi)
        mask = z > mid
        cnt = jnp.sum(mask.astype(jnp.float32), axis=1, keepdims=True)
        s = jnp.sum(jnp.where(mask, z, 0.0), axis=1, keepdims=True)
        pred = (s - cnt * mid) > _BUDGET
        lo = jnp.where(pred, mid, lo)
        hi = jnp.where(pred, hi, mid)
    tau = lo
    for _ in range(_N_CORRECT):
        mask = z > tau
        cnt = jnp.sum(mask.astype(jnp.float32), axis=1, keepdims=True)
        s = jnp.sum(jnp.where(mask, z, 0.0), axis=1, keepdims=True)
        tau = (s - _BUDGET) / jnp.maximum(cnt, 1.0)
    o_ref[...] = jnp.maximum(z - tau, 0.0) * (1.0 / _BUDGET)


@jax.jit
def kernel(input):
    bs, dim = input.shape
    rows_per_block = 8
    return pl.pallas_call(
        _sparsegen_block,
        grid=(bs // rows_per_block,),
        in_specs=[pl.BlockSpec((rows_per_block, dim), lambda i: (i, 0))],
        out_specs=pl.BlockSpec((rows_per_block, dim), lambda i: (i, 0)),
        out_shape=jax.ShapeDtypeStruct((bs, dim), jnp.float32),
    )(input.astype(jnp.float32))


# speedup vs baseline: 24.3196x; 1.1941x over previous
import jax
import jax.numpy as jnp
from jax.experimental import pallas as pl

_BUDGET = 1.0 - 0.05

_N_BISECT = 12
_N_CORRECT = 2


def _sparsegen_block(x_ref, o_ref):
    z = x_ref[...]
    rowmax = jnp.max(z, axis=1, keepdims=True)
    hi = rowmax
    lo = rowmax - _BUDGET
    for _ in range(_N_BISECT):
        mid = 0.5 * (lo + hi)
        r = jnp.sum(jnp.maximum(z - mid, 0.0), axis=1, keepdims=True)
        pred = r > _BUDGET
        lo = jnp.where(pred, mid, lo)
        hi = jnp.where(pred, hi, mid)
    tau = lo
    for _ in range(_N_CORRECT):
        mask = z > tau
        cnt = jnp.sum(mask.astype(jnp.float32), axis=1, keepdims=True)
        r = jnp.sum(jnp.maximum(z - tau, 0.0), axis=1, keepdims=True)
        tau = tau + (r - _BUDGET) / jnp.maximum(cnt, 1.0)
    o_ref[...] = jnp.maximum(z - tau, 0.0) * (1.0 / _BUDGET)


@jax.jit
def kernel(input):
    bs, dim = input.shape
    rows_per_block = 8
    return pl.pallas_call(
        _sparsegen_block,
        grid=(bs // rows_per_block,),
        in_specs=[pl.BlockSpec((rows_per_block, dim), lambda i: (i, 0))],
        out_specs=pl.BlockSpec((rows_per_block, dim), lambda i: (i, 0)),
        out_shape=jax.ShapeDtypeStruct((bs, dim), jnp.float32),
    )(input.astype(jnp.float32))


# speedup vs baseline: 44.7808x; 1.8413x over previous
import jax
import jax.numpy as jnp
from jax.experimental import pallas as pl

_BUDGET = 1.0 - 0.05

_N_BISECT = 10
_N_CORRECT = 2
_CHUNK = 2048


def _relu_sum(z, t, dim):
    acc = jnp.maximum(z[:, :_CHUNK] - t, 0.0)
    for i in range(1, dim // _CHUNK):
        acc = acc + jnp.maximum(z[:, i * _CHUNK:(i + 1) * _CHUNK] - t, 0.0)
    return jnp.sum(acc, axis=1, keepdims=True)


def _relu_sum_cnt(z, t, dim):
    acc = jnp.maximum(z[:, :_CHUNK] - t, 0.0)
    cacc = jnp.where(z[:, :_CHUNK] > t, 1.0, 0.0)
    for i in range(1, dim // _CHUNK):
        c = z[:, i * _CHUNK:(i + 1) * _CHUNK]
        acc = acc + jnp.maximum(c - t, 0.0)
        cacc = cacc + jnp.where(c > t, 1.0, 0.0)
    return (jnp.sum(acc, axis=1, keepdims=True),
            jnp.sum(cacc, axis=1, keepdims=True))


def _row_max(z, dim):
    acc = z[:, :_CHUNK]
    for i in range(1, dim // _CHUNK):
        acc = jnp.maximum(acc, z[:, i * _CHUNK:(i + 1) * _CHUNK])
    return jnp.max(acc, axis=1, keepdims=True)


def _sparsegen_block(x_ref, o_ref):
    z = x_ref[...]
    dim = z.shape[1]
    rowmax = _row_max(z, dim)
    hi = rowmax
    lo = rowmax - _BUDGET
    for _ in range(_N_BISECT):
        mid = 0.5 * (lo + hi)
        r = _relu_sum(z, mid, dim)
        pred = r > _BUDGET
        lo = jnp.where(pred, mid, lo)
        hi = jnp.where(pred, hi, mid)
    tau = lo
    for _ in range(_N_CORRECT):
        r, cnt = _relu_sum_cnt(z, tau, dim)
        tau = tau + (r - _BUDGET) / jnp.maximum(cnt, 1.0)
    o_ref[...] = jnp.maximum(z - tau, 0.0) * (1.0 / _BUDGET)


@jax.jit
def kernel(input):
    bs, dim = input.shape
    rows_per_block = 8
    return pl.pallas_call(
        _sparsegen_block,
        grid=(bs // rows_per_block,),
        in_specs=[pl.BlockSpec((rows_per_block, dim), lambda i: (i, 0))],
        out_specs=pl.BlockSpec((rows_per_block, dim), lambda i: (i, 0)),
        out_shape=jax.ShapeDtypeStruct((bs, dim), jnp.float32),
    )(input.astype(jnp.float32))


# speedup vs baseline: 45.3987x; 1.0138x over previous
import jax
import jax.numpy as jnp
from jax.experimental import pallas as pl

_BUDGET = 1.0 - 0.05

_N_BISECT = 8
_N_CORRECT = 3
_CHUNK = 2048


def _relu_sum(z, t, dim):
    acc = jnp.maximum(z[:, :_CHUNK] - t, 0.0)
    for i in range(1, dim // _CHUNK):
        acc = acc + jnp.maximum(z[:, i * _CHUNK:(i + 1) * _CHUNK] - t, 0.0)
    return jnp.sum(acc, axis=1, keepdims=True)


def _relu_sum_cnt(z, t, dim):
    acc = jnp.maximum(z[:, :_CHUNK] - t, 0.0)
    cacc = jnp.where(z[:, :_CHUNK] > t, 1.0, 0.0)
    for i in range(1, dim // _CHUNK):
        c = z[:, i * _CHUNK:(i + 1) * _CHUNK]
        acc = acc + jnp.maximum(c - t, 0.0)
        cacc = cacc + jnp.where(c > t, 1.0, 0.0)
    return (jnp.sum(acc, axis=1, keepdims=True),
            jnp.sum(cacc, axis=1, keepdims=True))


def _row_max(z, dim):
    acc = z[:, :_CHUNK]
    for i in range(1, dim // _CHUNK):
        acc = jnp.maximum(acc, z[:, i * _CHUNK:(i + 1) * _CHUNK])
    return jnp.max(acc, axis=1, keepdims=True)


def _sparsegen_block(x_ref, o_ref):
    z = x_ref[...]
    dim = z.shape[1]
    zh = z.astype(jnp.bfloat16)
    rowmax = _row_max(z, dim)
    hi = rowmax
    lo = rowmax - _BUDGET
    for _ in range(_N_BISECT):
        mid = 0.5 * (lo + hi)
        r = _relu_sum(zh, mid.astype(jnp.bfloat16), dim).astype(jnp.float32)
        pred = r > _BUDGET
        lo = jnp.where(pred, mid, lo)
        hi = jnp.where(pred, hi, mid)
    tau = lo
    for _ in range(_N_CORRECT):
        r, cnt = _relu_sum_cnt(z, tau, dim)
        tau = tau + (r - _BUDGET) / jnp.maximum(cnt, 1.0)
    o_ref[...] = jnp.maximum(z - tau, 0.0) * (1.0 / _BUDGET)


@jax.jit
def kernel(input):
    bs, dim = input.shape
    rows_per_block = 8
    return pl.pallas_call(
        _sparsegen_block,
        grid=(bs // rows_per_block,),
        in_specs=[pl.BlockSpec((rows_per_block, dim), lambda i: (i, 0))],
        out_specs=pl.BlockSpec((rows_per_block, dim), lambda i: (i, 0)),
        out_shape=jax.ShapeDtypeStruct((bs, dim), jnp.float32),
    )(input.astype(jnp.float32))


# speedup vs baseline: 75.5643x; 1.6645x over previous
import jax
import jax.numpy as jnp
from jax.experimental import pallas as pl

_BUDGET = 1.0 - 0.05

_N_BISECT = 10
_N_CORRECT = 1
_CHUNK = 2048


def _relu_sum(z, t, dim):
    acc = jnp.maximum(z[:, :_CHUNK] - t, 0.0)
    for i in range(1, dim // _CHUNK):
        acc = acc + jnp.maximum(z[:, i * _CHUNK:(i + 1) * _CHUNK] - t, 0.0)
    return jnp.sum(acc, axis=1, keepdims=True)


def _relu_sum_cnt(z, t, dim):
    acc = jnp.maximum(z[:, :_CHUNK] - t, 0.0)
    cacc = jnp.where(z[:, :_CHUNK] > t, 1.0, 0.0)
    for i in range(1, dim // _CHUNK):
        c = z[:, i * _CHUNK:(i + 1) * _CHUNK]
        acc = acc + jnp.maximum(c - t, 0.0)
        cacc = cacc + jnp.where(c > t, 1.0, 0.0)
    return (jnp.sum(acc, axis=1, keepdims=True),
            jnp.sum(cacc, axis=1, keepdims=True))


def _row_max(z, dim):
    acc = z[:, :_CHUNK]
    for i in range(1, dim // _CHUNK):
        acc = jnp.maximum(acc, z[:, i * _CHUNK:(i + 1) * _CHUNK])
    return jnp.max(acc, axis=1, keepdims=True)


def _sparsegen_block(x_ref, o_ref):
    z = x_ref[...]
    dim = z.shape[1]
    rowmax = _row_max(z, dim)
    hi = rowmax
    lo = rowmax - _BUDGET
    for _ in range(_N_BISECT):
        mid = 0.5 * (lo + hi)
        r = _relu_sum(z, mid, dim)
        pred = r > _BUDGET
        lo = jnp.where(pred, mid, lo)
        hi = jnp.where(pred, hi, mid)
    tau = lo
    for _ in range(_N_CORRECT):
        r, cnt = _relu_sum_cnt(z, tau, dim)
        tau = tau + (r - _BUDGET) / jnp.maximum(cnt, 1.0)
    o_ref[...] = jnp.maximum(z - tau, 0.0) * (1.0 / _BUDGET)


@jax.jit
def kernel(input):
    bs, dim = input.shape
    rows_per_block = 16
    return pl.pallas_call(
        _sparsegen_block,
        grid=(bs // rows_per_block,),
        in_specs=[pl.BlockSpec((rows_per_block, dim), lambda i: (i, 0))],
        out_specs=pl.BlockSpec((rows_per_block, dim), lambda i: (i, 0)),
        out_shape=jax.ShapeDtypeStruct((bs, dim), jnp.float32),
    )(input.astype(jnp.float32))


# speedup vs baseline: 83.5938x; 1.1063x over previous
import jax
import jax.numpy as jnp
from jax.experimental import pallas as pl

_BUDGET = 1.0 - 0.05

_N_BISECT = 8
_CHUNK = 2048


def _relu_sum(z, t, dim):
    acc = jnp.maximum(z[:, :_CHUNK] - t, 0.0)
    for i in range(1, dim // _CHUNK):
        acc = acc + jnp.maximum(z[:, i * _CHUNK:(i + 1) * _CHUNK] - t, 0.0)
    return jnp.sum(acc, axis=1, keepdims=True)


def _relu_sum_cnt(z, t, dim):
    acc = jnp.maximum(z[:, :_CHUNK] - t, 0.0)
    cacc = jnp.where(z[:, :_CHUNK] > t, 1.0, 0.0)
    for i in range(1, dim // _CHUNK):
        c = z[:, i * _CHUNK:(i + 1) * _CHUNK]
        acc = acc + jnp.maximum(c - t, 0.0)
        cacc = cacc + jnp.where(c > t, 1.0, 0.0)
    return (jnp.sum(acc, axis=1, keepdims=True),
            jnp.sum(cacc, axis=1, keepdims=True))


def _row_max(z, dim):
    acc = z[:, :_CHUNK]
    for i in range(1, dim // _CHUNK):
        acc = jnp.maximum(acc, z[:, i * _CHUNK:(i + 1) * _CHUNK])
    return jnp.max(acc, axis=1, keepdims=True)


def _sparsegen_block(x_ref, o_ref):
    z = x_ref[...]
    dim = z.shape[1]
    rowmax = _row_max(z, dim)
    hi = rowmax
    lo = rowmax - _BUDGET
    for _ in range(_N_BISECT):
        mid = 0.5 * (lo + hi)
        r = _relu_sum(z, mid, dim)
        pred = r > _BUDGET
        lo = jnp.where(pred, mid, lo)
        hi = jnp.where(pred, hi, mid)
    mid = 0.5 * (lo + hi)
    r, cnt = _relu_sum_cnt(z, mid, dim)
    tau = mid + (r - _BUDGET) / jnp.maximum(cnt, 1.0)
    o_ref[...] = jnp.maximum(z - tau, 0.0) * (1.0 / _BUDGET)


@jax.jit
def kernel(input):
    bs, dim = input.shape
    rows_per_block = 32
    return pl.pallas_call(
        _sparsegen_block,
        grid=(bs // rows_per_block,),
        in_specs=[pl.BlockSpec((rows_per_block, dim), lambda i: (i, 0))],
        out_specs=pl.BlockSpec((rows_per_block, dim), lambda i: (i, 0)),
        out_shape=jax.ShapeDtypeStruct((bs, dim), jnp.float32),
    )(input.astype(jnp.float32))


# speedup vs baseline: 98.3238x; 1.1762x over previous
import jax
import jax.numpy as jnp
from jax.experimental import pallas as pl

_BUDGET = 1.0 - 0.05

_N_BISECT = 8
_CHUNK = 2048


def _relu_sum(z, t, dim):
    acc = jnp.maximum(z[:, :_CHUNK] - t, 0.0)
    for i in range(1, dim // _CHUNK):
        acc = acc + jnp.maximum(z[:, i * _CHUNK:(i + 1) * _CHUNK] - t, 0.0)
    return jnp.sum(acc, axis=1, keepdims=True)


def _relu_sum_cnt(z, t, dim):
    acc = jnp.maximum(z[:, :_CHUNK] - t, 0.0)
    cacc = jnp.where(z[:, :_CHUNK] > t, 1.0, 0.0)
    for i in range(1, dim // _CHUNK):
        c = z[:, i * _CHUNK:(i + 1) * _CHUNK]
        acc = acc + jnp.maximum(c - t, 0.0)
        cacc = cacc + jnp.where(c > t, 1.0, 0.0)
    return (jnp.sum(acc, axis=1, keepdims=True),
            jnp.sum(cacc, axis=1, keepdims=True))


def _row_max(z, dim):
    acc = z[:, :_CHUNK]
    for i in range(1, dim // _CHUNK):
        acc = jnp.maximum(acc, z[:, i * _CHUNK:(i + 1) * _CHUNK])
    return jnp.max(acc, axis=1, keepdims=True)


def _sparsegen_block(x_ref, o_ref):
    z = x_ref[...]
    dim = z.shape[1]
    rowmax = _row_max(z, dim)
    w = (z - rowmax).astype(jnp.bfloat16)
    lo = jnp.full(rowmax.shape, -_BUDGET, jnp.float32)
    hi = jnp.zeros(rowmax.shape, jnp.float32)
    for _ in range(_N_BISECT):
        mid = 0.5 * (lo + hi)
        r = _relu_sum(w, mid.astype(jnp.bfloat16), dim).astype(jnp.float32)
        pred = r > _BUDGET
        lo = jnp.where(pred, mid, lo)
        hi = jnp.where(pred, hi, mid)
    t = rowmax + 0.5 * (lo + hi)
    r, cnt = _relu_sum_cnt(z, t, dim)
    tau = t + (r - _BUDGET) / jnp.maximum(cnt, 1.0)
    o_ref[...] = jnp.maximum(z - tau, 0.0) * (1.0 / _BUDGET)


@jax.jit
def kernel(input):
    bs, dim = input.shape
    rows_per_block = 32
    return pl.pallas_call(
        _sparsegen_block,
        grid=(bs // rows_per_block,),
        in_specs=[pl.BlockSpec((rows_per_block, dim), lambda i: (i, 0))],
        out_specs=pl.BlockSpec((rows_per_block, dim), lambda i: (i, 0)),
        out_shape=jax.ShapeDtypeStruct((bs, dim), jnp.float32),
    )(input.astype(jnp.float32))
